# baseline (device time: 761819 ns/iter reference)
import jax
import jax.numpy as jnp
from jax import lax
from jax.experimental import pallas as pl
from jax.experimental.pallas import tpu as pltpu

N_DEV = 16


def kernel(x, w_mat):
    m_per, k = x.shape
    _, n_per = w_mat.shape

    def body(x_ref, w_ref, out_ref, comm_ref, send_sems, recv_sems, credit_sem):
        my = lax.axis_index("i")
        left = lax.rem(my + N_DEV - 1, N_DEV)
        right = lax.rem(my + 1, N_DEV)

        barrier_sem = pltpu.get_barrier_semaphore()
        for nbr in (left, right):
            pl.semaphore_signal(
                barrier_sem, inc=1,
                device_id=(nbr,), device_id_type=pl.DeviceIdType.MESH,
            )
        pl.semaphore_wait(barrier_sem, 2)

        comm_ref[0] = x_ref[...]
        y = jnp.dot(x_ref[...], w_ref[...], preferred_element_type=jnp.float32)
        out_ref[pl.ds(my * m_per, m_per), :] = y * jax.nn.sigmoid(y)

        for h in range(N_DEV - 1):
            s = h % 2
            r = (h + 1) % 2
            if h >= 1:
                pl.semaphore_wait(credit_sem, 1)
            rdma = pltpu.make_async_remote_copy(
                src_ref=comm_ref.at[s],
                dst_ref=comm_ref.at[r],
                send_sem=send_sems.at[s],
                recv_sem=recv_sems.at[r],
                device_id=(right,),
                device_id_type=pl.DeviceIdType.MESH,
            )
            rdma.start()
            rdma.wait()
            if h < N_DEV - 2:
                pl.semaphore_signal(
                    credit_sem, inc=1,
                    device_id=(left,), device_id_type=pl.DeviceIdType.MESH,
                )
            origin = lax.rem(my + N_DEV - h - 1, N_DEV)
            xc = comm_ref[r]
            y = jnp.dot(xc, w_ref[...], preferred_element_type=jnp.float32)
            out_ref[pl.ds(origin * m_per, m_per), :] = y * jax.nn.sigmoid(y)

    return pl.pallas_call(
        body,
        out_shape=jax.ShapeDtypeStruct((N_DEV * m_per, n_per), jnp.float32),
        in_specs=[
            pl.BlockSpec(memory_space=pltpu.VMEM),
            pl.BlockSpec(memory_space=pltpu.VMEM),
        ],
        out_specs=pl.BlockSpec(memory_space=pltpu.VMEM),
        scratch_shapes=[
            pltpu.VMEM((2, m_per, k), jnp.float32),
            pltpu.SemaphoreType.DMA((2,)),
            pltpu.SemaphoreType.DMA((2,)),
            pltpu.SemaphoreType.REGULAR,
        ],
        compiler_params=pltpu.CompilerParams(collective_id=0),
    )(x, w_mat)


# device time: 410482 ns/iter; 1.8559x vs baseline; 1.8559x over previous
import jax
import jax.numpy as jnp
from jax import lax
from jax.experimental import pallas as pl
from jax.experimental.pallas import tpu as pltpu

N_DEV = 16
HOPS_R = 8
HOPS_L = 7


def kernel(x, w_mat):
    m_per, k = x.shape
    _, n_per = w_mat.shape

    def body(x_ref, w_ref, out_ref,
             comm_r, comm_l,
             send_r, recv_r, send_l, recv_l,
             credit_r, credit_l):
        my = lax.axis_index("i")
        left = lax.rem(my + N_DEV - 1, N_DEV)
        right = lax.rem(my + 1, N_DEV)

        def compute(origin, chunk):
            y = jnp.dot(chunk, w_ref[...], preferred_element_type=jnp.float32)
            out_ref[pl.ds(origin * m_per, m_per), :] = y * jax.nn.sigmoid(y)

        barrier_sem = pltpu.get_barrier_semaphore()
        for nbr in (left, right):
            pl.semaphore_signal(
                barrier_sem, inc=1,
                device_id=(nbr,), device_id_type=pl.DeviceIdType.MESH,
            )
        pl.semaphore_wait(barrier_sem, 2)

        comm_r[0] = x_ref[...]
        comm_l[0] = x_ref[...]

        for j in range(HOPS_R):
            s = j % 2
            r = (j + 1) % 2
            if j >= 1:
                pl.semaphore_wait(credit_r, 1)
            rdma_r = pltpu.make_async_remote_copy(
                src_ref=comm_r.at[s], dst_ref=comm_r.at[r],
                send_sem=send_r.at[s], recv_sem=recv_r.at[r],
                device_id=(right,), device_id_type=pl.DeviceIdType.MESH,
            )
            rdma_r.start()
            if j < HOPS_L:
                if j >= 1:
                    pl.semaphore_wait(credit_l, 1)
                rdma_l = pltpu.make_async_remote_copy(
                    src_ref=comm_l.at[s], dst_ref=comm_l.at[r],
                    send_sem=send_l.at[s], recv_sem=recv_l.at[r],
                    device_id=(left,), device_id_type=pl.DeviceIdType.MESH,
                )
                rdma_l.start()

            if j == 0:
                compute(my, x_ref[...])
            else:
                compute(lax.rem(my + N_DEV - j, N_DEV), comm_r[s])
                compute(lax.rem(my + j, N_DEV), comm_l[s])

            rdma_r.wait()
            if j < HOPS_R - 1:
                pl.semaphore_signal(
                    credit_r, inc=1,
                    device_id=(left,), device_id_type=pl.DeviceIdType.MESH,
                )
            if j < HOPS_L:
                rdma_l.wait()
                if j < HOPS_L - 1:
                    pl.semaphore_signal(
                        credit_l, inc=1,
                        device_id=(right,), device_id_type=pl.DeviceIdType.MESH,
                    )

        compute(lax.rem(my + N_DEV - HOPS_R, N_DEV), comm_r[HOPS_R % 2])
        compute(lax.rem(my + HOPS_L, N_DEV), comm_l[HOPS_L % 2])

    return pl.pallas_call(
        body,
        out_shape=jax.ShapeDtypeStruct((N_DEV * m_per, n_per), jnp.float32),
        in_specs=[
            pl.BlockSpec(memory_space=pltpu.VMEM),
            pl.BlockSpec(memory_space=pltpu.VMEM),
        ],
        out_specs=pl.BlockSpec(memory_space=pltpu.VMEM),
        scratch_shapes=[
            pltpu.VMEM((2, m_per, k), jnp.float32),
            pltpu.VMEM((2, m_per, k), jnp.float32),
            pltpu.SemaphoreType.DMA((2,)),
            pltpu.SemaphoreType.DMA((2,)),
            pltpu.SemaphoreType.DMA((2,)),
            pltpu.SemaphoreType.DMA((2,)),
            pltpu.SemaphoreType.REGULAR,
            pltpu.SemaphoreType.REGULAR,
        ],
        compiler_params=pltpu.CompilerParams(collective_id=0),
    )(x, w_mat)


# device time: 387991 ns/iter; 1.9635x vs baseline; 1.0580x over previous
import jax
import jax.numpy as jnp
from jax import lax
from jax.experimental import pallas as pl
from jax.experimental.pallas import tpu as pltpu

N_DEV = 16
HOPS = 8
HALF = 128


def kernel(x, w_mat):
    m_per, k = x.shape
    _, n_per = w_mat.shape

    def body(x_ref, w_ref, out_ref,
             comm_r, comm_l,
             send_r, recv_r, send_l, recv_l,
             credit_r, credit_l):
        my = lax.axis_index("i")
        left = lax.rem(my + N_DEV - 1, N_DEV)
        right = lax.rem(my + 1, N_DEV)

        def compute(origin, chunk, row0=0, rows=m_per):
            y = jnp.dot(chunk, w_ref[...], preferred_element_type=jnp.float32)
            out_ref[pl.ds(origin * m_per + row0, rows), :] = y * jax.nn.sigmoid(y)

        barrier_sem = pltpu.get_barrier_semaphore()
        for nbr in (left, right):
            pl.semaphore_signal(
                barrier_sem, inc=1,
                device_id=(nbr,), device_id_type=pl.DeviceIdType.MESH,
            )
        pl.semaphore_wait(barrier_sem, 2)

        comm_r[0] = x_ref[...]
        comm_l[0] = x_ref[...]

        for j in range(HOPS):
            s = j % 2
            r = (j + 1) % 2
            lastj = j == HOPS - 1
            if j >= 1:
                pl.semaphore_wait(credit_r, 1)
            rdma_r = pltpu.make_async_remote_copy(
                src_ref=comm_r.at[s, pl.ds(0, HALF)] if lastj else comm_r.at[s],
                dst_ref=comm_r.at[r, pl.ds(0, HALF)] if lastj else comm_r.at[r],
                send_sem=send_r.at[s], recv_sem=recv_r.at[r],
                device_id=(right,), device_id_type=pl.DeviceIdType.MESH,
            )
            rdma_r.start()
            if j >= 1:
                pl.semaphore_wait(credit_l, 1)
            rdma_l = pltpu.make_async_remote_copy(
                src_ref=comm_l.at[s, pl.ds(HALF, HALF)] if lastj else comm_l.at[s],
                dst_ref=comm_l.at[r, pl.ds(HALF, HALF)] if lastj else comm_l.at[r],
                send_sem=send_l.at[s], recv_sem=recv_l.at[r],
                device_id=(left,), device_id_type=pl.DeviceIdType.MESH,
            )
            rdma_l.start()

            if j == 0:
                compute(my, x_ref[...])
            else:
                compute(lax.rem(my + N_DEV - j, N_DEV), comm_r[s])
                compute(lax.rem(my + j, N_DEV), comm_l[s])

            rdma_r.wait()
            if not lastj:
                pl.semaphore_signal(
                    credit_r, inc=1,
                    device_id=(left,), device_id_type=pl.DeviceIdType.MESH,
                )
            rdma_l.wait()
            if not lastj:
                pl.semaphore_signal(
                    credit_l, inc=1,
                    device_id=(right,), device_id_type=pl.DeviceIdType.MESH,
                )

        anti = lax.rem(my + N_DEV // 2, N_DEV)
        compute(anti, comm_r[0, pl.ds(0, HALF)], row0=0, rows=HALF)
        compute(anti, comm_l[0, pl.ds(HALF, HALF)], row0=HALF, rows=HALF)

    return pl.pallas_call(
        body,
        out_shape=jax.ShapeDtypeStruct((N_DEV * m_per, n_per), jnp.float32),
        in_specs=[
            pl.BlockSpec(memory_space=pltpu.VMEM),
            pl.BlockSpec(memory_space=pltpu.VMEM),
        ],
        out_specs=pl.BlockSpec(memory_space=pltpu.VMEM),
        scratch_shapes=[
            pltpu.VMEM((2, m_per, k), jnp.float32),
            pltpu.VMEM((2, m_per, k), jnp.float32),
            pltpu.SemaphoreType.DMA((2,)),
            pltpu.SemaphoreType.DMA((2,)),
            pltpu.SemaphoreType.DMA((2,)),
            pltpu.SemaphoreType.DMA((2,)),
            pltpu.SemaphoreType.REGULAR,
            pltpu.SemaphoreType.REGULAR,
        ],
        compiler_params=pltpu.CompilerParams(collective_id=0),
    )(x, w_mat)


# device time: 376047 ns/iter; 2.0259x vs baseline; 1.0318x over previous
import jax
import jax.numpy as jnp
from jax import lax
from jax.experimental import pallas as pl
from jax.experimental.pallas import tpu as pltpu

N_DEV = 16
HOPS = 8
HALF = 128
NSLOT = 4


def kernel(x, w_mat):
    m_per, k = x.shape
    _, n_per = w_mat.shape

    def body(x_ref, w_ref, out_ref,
             comm_r, comm_l,
             send_r, recv_r, send_l, recv_l,
             credit_r, credit_l):
        my = lax.axis_index("i")
        left = lax.rem(my + N_DEV - 1, N_DEV)
        right = lax.rem(my + 1, N_DEV)

        def compute(origin, chunk, row0=0, rows=m_per):
            y = jnp.dot(chunk, w_ref[...], preferred_element_type=jnp.float32)
            out_ref[pl.ds(origin * m_per + row0, rows), :] = y * jax.nn.sigmoid(y)

        def hop_rdma(j, dirn):
            comm = comm_r if dirn == 0 else comm_l
            s, r = j % NSLOT, (j + 1) % NSLOT
            if j == HOPS - 1:
                rows = pl.ds(0, HALF) if dirn == 0 else pl.ds(HALF, HALF)
                src, dst = comm.at[s, rows], comm.at[r, rows]
            elif j == 0:
                src, dst = x_ref, comm.at[r]
            else:
                src, dst = comm.at[s], comm.at[r]
            return pltpu.make_async_remote_copy(
                src_ref=src, dst_ref=dst,
                send_sem=(send_r if dirn == 0 else send_l).at[s],
                recv_sem=(recv_r if dirn == 0 else recv_l).at[r],
                device_id=(right,) if dirn == 0 else (left,),
                device_id_type=pl.DeviceIdType.MESH,
            )

        barrier_sem = pltpu.get_barrier_semaphore()
        for nbr in (left, right):
            pl.semaphore_signal(
                barrier_sem, inc=1,
                device_id=(nbr,), device_id_type=pl.DeviceIdType.MESH,
            )
        pl.semaphore_wait(barrier_sem, 2)

        prev_r = prev_l = None
        for j in range(HOPS):
            if j >= 1:
                prev_r.wait_send()
                if 2 <= j <= 5:
                    pl.semaphore_signal(
                        credit_r, inc=1,
                        device_id=(left,), device_id_type=pl.DeviceIdType.MESH,
                    )
                prev_l.wait_send()
                if 2 <= j <= 5:
                    pl.semaphore_signal(
                        credit_l, inc=1,
                        device_id=(right,), device_id_type=pl.DeviceIdType.MESH,
                    )
            if j >= 4:
                pl.semaphore_wait(credit_r, 1)
            rdma_r = hop_rdma(j, 0)
            rdma_r.start()
            if j >= 4:
                pl.semaphore_wait(credit_l, 1)
            rdma_l = hop_rdma(j, 1)
            rdma_l.start()

            if j == 0:
                compute(my, x_ref[...])
            else:
                compute(lax.rem(my + N_DEV - j, N_DEV), comm_r[j % NSLOT])
                compute(lax.rem(my + j, N_DEV), comm_l[j % NSLOT])

            rdma_r.wait_recv()
            rdma_l.wait_recv()
            prev_r, prev_l = rdma_r, rdma_l

        prev_r.wait_send()
        prev_l.wait_send()

        anti = lax.rem(my + N_DEV // 2, N_DEV)
        compute(anti, comm_r[0, pl.ds(0, HALF)], row0=0, rows=HALF)
        compute(anti, comm_l[0, pl.ds(HALF, HALF)], row0=HALF, rows=HALF)

    return pl.pallas_call(
        body,
        out_shape=jax.ShapeDtypeStruct((N_DEV * m_per, n_per), jnp.float32),
        in_specs=[
            pl.BlockSpec(memory_space=pltpu.VMEM),
            pl.BlockSpec(memory_space=pltpu.VMEM),
        ],
        out_specs=pl.BlockSpec(memory_space=pltpu.VMEM),
        scratch_shapes=[
            pltpu.VMEM((NSLOT, m_per, k), jnp.float32),
            pltpu.VMEM((NSLOT, m_per, k), jnp.float32),
            pltpu.SemaphoreType.DMA((NSLOT,)),
            pltpu.SemaphoreType.DMA((NSLOT,)),
            pltpu.SemaphoreType.DMA((NSLOT,)),
            pltpu.SemaphoreType.DMA((NSLOT,)),
            pltpu.SemaphoreType.REGULAR,
            pltpu.SemaphoreType.REGULAR,
        ],
        compiler_params=pltpu.CompilerParams(
            collective_id=0, vmem_limit_bytes=100 * 1024 * 1024
        ),
    )(x, w_mat)


# device time: 357911 ns/iter; 2.1285x vs baseline; 1.0507x over previous
import jax
import jax.numpy as jnp
from jax import lax
from jax.experimental import pallas as pl
from jax.experimental.pallas import tpu as pltpu

N_DEV = 16
HOPS = 8
HALF = 128
NSLOT = 4


def kernel(x, w_mat):
    m_per, k = x.shape
    _, n_per = w_mat.shape

    def body(x_ref, w_ref, out_ref,
             comm_r, comm_l,
             send_r, recv_r, send_l, recv_l,
             credit_r, credit_l):
        my = lax.axis_index("i")
        left = lax.rem(my + N_DEV - 1, N_DEV)
        right = lax.rem(my + 1, N_DEV)

        def compute(origin, chunk, row0=0, rows=m_per):
            y = jnp.dot(chunk, w_ref[...], preferred_element_type=jnp.float32)
            out_ref[pl.ds(origin * m_per + row0, rows), :] = y * jax.nn.sigmoid(y)

        def pieces(j, dirn):
            if j < HOPS - 1:
                return (0, 1)
            return (0,) if dirn == 0 else (1,)

        def mk(j, p, dirn):
            comm = comm_r if dirn == 0 else comm_l
            rows = pl.ds(p * HALF, HALF)
            src = x_ref.at[rows] if j == 0 else comm.at[j % NSLOT, rows]
            return pltpu.make_async_remote_copy(
                src_ref=src,
                dst_ref=comm.at[(j + 1) % NSLOT, rows],
                send_sem=(send_r if dirn == 0 else send_l).at[j % NSLOT, p],
                recv_sem=(recv_r if dirn == 0 else recv_l).at[(j + 1) % NSLOT, p],
                device_id=(right,) if dirn == 0 else (left,),
                device_id_type=pl.DeviceIdType.MESH,
            )

        barrier_sem = pltpu.get_barrier_semaphore()
        for nbr in (left, right):
            pl.semaphore_signal(
                barrier_sem, inc=1,
                device_id=(nbr,), device_id_type=pl.DeviceIdType.MESH,
            )
        pl.semaphore_wait(barrier_sem, 2)

        descs = {}
        for j in range(HOPS):
            if j >= 1:
                for p in pieces(j - 1, 0):
                    descs[0, j - 1, p].wait_send()
                if 2 <= j <= 5:
                    pl.semaphore_signal(
                        credit_r, inc=1,
                        device_id=(left,), device_id_type=pl.DeviceIdType.MESH,
                    )
                for p in pieces(j - 1, 1):
                    descs[1, j - 1, p].wait_send()
                if 2 <= j <= 5:
                    pl.semaphore_signal(
                        credit_l, inc=1,
                        device_id=(right,), device_id_type=pl.DeviceIdType.MESH,
                    )
            if j >= 4:
                pl.semaphore_wait(credit_r, 1)
                pl.semaphore_wait(credit_l, 1)

            if 0 in pieces(j, 0):
                descs[0, j, 0] = mk(j, 0, 0)
                descs[0, j, 0].start()
            if 0 in pieces(j, 1):
                descs[1, j, 0] = mk(j, 0, 1)
                descs[1, j, 0].start()

            if j >= 1:
                descs[0, j - 1, 1].wait_recv()
            if 1 in pieces(j, 0):
                descs[0, j, 1] = mk(j, 1, 0)
                descs[0, j, 1].start()
            if j >= 1:
                descs[1, j - 1, 1].wait_recv()
            if 1 in pieces(j, 1):
                descs[1, j, 1] = mk(j, 1, 1)
                descs[1, j, 1].start()

            if j == 0:
                compute(my, x_ref[...])
            else:
                compute(lax.rem(my + N_DEV - j, N_DEV), comm_r[j % NSLOT])
                compute(lax.rem(my + j, N_DEV), comm_l[j % NSLOT])

            if 0 in pieces(j, 0):
                descs[0, j, 0].wait_recv()
            if 0 in pieces(j, 1):
                descs[1, j, 0].wait_recv()

        descs[1, HOPS - 1, 1].wait_recv()
        descs[0, HOPS - 1, 0].wait_send()
        descs[1, HOPS - 1, 1].wait_send()

        anti = lax.rem(my + N_DEV // 2, N_DEV)
        compute(anti, comm_r[0, pl.ds(0, HALF)], row0=0, rows=HALF)
        compute(anti, comm_l[0, pl.ds(HALF, HALF)], row0=HALF, rows=HALF)

    return pl.pallas_call(
        body,
        out_shape=jax.ShapeDtypeStruct((N_DEV * m_per, n_per), jnp.float32),
        in_specs=[
            pl.BlockSpec(memory_space=pltpu.VMEM),
            pl.BlockSpec(memory_space=pltpu.VMEM),
        ],
        out_specs=pl.BlockSpec(memory_space=pltpu.VMEM),
        scratch_shapes=[
            pltpu.VMEM((NSLOT, m_per, k), jnp.float32),
            pltpu.VMEM((NSLOT, m_per, k), jnp.float32),
            pltpu.SemaphoreType.DMA((NSLOT, 2)),
            pltpu.SemaphoreType.DMA((NSLOT, 2)),
            pltpu.SemaphoreType.DMA((NSLOT, 2)),
            pltpu.SemaphoreType.DMA((NSLOT, 2)),
            pltpu.SemaphoreType.REGULAR,
            pltpu.SemaphoreType.REGULAR,
        ],
        compiler_params=pltpu.CompilerParams(
            collective_id=0, vmem_limit_bytes=100 * 1024 * 1024
        ),
    )(x, w_mat)


# device time: 357478 ns/iter; 2.1311x vs baseline; 1.0012x over previous
import jax
import jax.numpy as jnp
from jax import lax
from jax.experimental import pallas as pl
from jax.experimental.pallas import tpu as pltpu

N_DEV = 16
HOPS = 8
HALF = 128
NSLOT = 4


def kernel(x, w_mat):
    m_per, k = x.shape
    _, n_per = w_mat.shape

    def body(x_ref, w_ref, out_ref,
             comm_r, comm_l,
             send_r, recv_r, send_l, recv_l,
             credit_r, credit_l):
        my = lax.axis_index("i")
        left = lax.rem(my + N_DEV - 1, N_DEV)
        right = lax.rem(my + 1, N_DEV)

        def compute(origin, chunk, row0=0, rows=m_per):
            y = jnp.dot(chunk, w_ref[...], preferred_element_type=jnp.float32)
            out_ref[pl.ds(origin * m_per + row0, rows), :] = y * jax.nn.sigmoid(y)

        def pieces(j, dirn):
            if j < HOPS - 1:
                return (0, 1)
            return (0,) if dirn == 0 else (1,)

        def mk(j, p, dirn):
            comm = comm_r if dirn == 0 else comm_l
            rows = pl.ds(p * HALF, HALF)
            src = x_ref.at[rows] if j == 0 else comm.at[j % NSLOT, rows]
            return pltpu.make_async_remote_copy(
                src_ref=src,
                dst_ref=comm.at[(j + 1) % NSLOT, rows],
                send_sem=(send_r if dirn == 0 else send_l).at[j % NSLOT, p],
                recv_sem=(recv_r if dirn == 0 else recv_l).at[(j + 1) % NSLOT, p],
                device_id=(right,) if dirn == 0 else (left,),
                device_id_type=pl.DeviceIdType.MESH,
            )

        barrier_sem = pltpu.get_barrier_semaphore()
        for nbr in (left, right):
            pl.semaphore_signal(
                barrier_sem, inc=1,
                device_id=(nbr,), device_id_type=pl.DeviceIdType.MESH,
            )
        pl.semaphore_wait(barrier_sem, 2)

        descs = {}
        for j in range(HOPS):
            if j >= 1:
                for p in pieces(j - 1, 0):
                    descs[0, j - 1, p].wait_send()
                if 2 <= j <= 5:
                    pl.semaphore_signal(
                        credit_r, inc=1,
                        device_id=(left,), device_id_type=pl.DeviceIdType.MESH,
                    )
                for p in pieces(j - 1, 1):
                    descs[1, j - 1, p].wait_send()
                if 2 <= j <= 5:
                    pl.semaphore_signal(
                        credit_l, inc=1,
                        device_id=(right,), device_id_type=pl.DeviceIdType.MESH,
                    )
            if j >= 4:
                pl.semaphore_wait(credit_r, 1)
                pl.semaphore_wait(credit_l, 1)

            if 0 in pieces(j, 0):
                descs[0, j, 0] = mk(j, 0, 0)
                descs[0, j, 0].start()
            if 0 in pieces(j, 1):
                descs[1, j, 0] = mk(j, 0, 1)
                descs[1, j, 0].start()

            if j >= 1:
                descs[0, j - 1, 1].wait_recv()
            if 1 in pieces(j, 0):
                descs[0, j, 1] = mk(j, 1, 0)
                descs[0, j, 1].start()
            if j >= 1:
                descs[1, j - 1, 1].wait_recv()
            if 1 in pieces(j, 1):
                descs[1, j, 1] = mk(j, 1, 1)
                descs[1, j, 1].start()

            if j == 0:
                compute(my, x_ref[...])
            else:
                compute(lax.rem(my + N_DEV - j, N_DEV), comm_r[j % NSLOT])
                compute(lax.rem(my + j, N_DEV), comm_l[j % NSLOT])

            if 0 in pieces(j, 0):
                descs[0, j, 0].wait_recv()
                if j == HOPS - 1:
                    anti = lax.rem(my + N_DEV // 2, N_DEV)
                    compute(anti, comm_r[0, pl.ds(0, HALF)], row0=0, rows=HALF)
            if 0 in pieces(j, 1):
                descs[1, j, 0].wait_recv()

        descs[1, HOPS - 1, 1].wait_recv()
        compute(anti, comm_l[0, pl.ds(HALF, HALF)], row0=HALF, rows=HALF)
        descs[0, HOPS - 1, 0].wait_send()
        descs[1, HOPS - 1, 1].wait_send()

    return pl.pallas_call(
        body,
        out_shape=jax.ShapeDtypeStruct((N_DEV * m_per, n_per), jnp.float32),
        in_specs=[
            pl.BlockSpec(memory_space=pltpu.VMEM),
            pl.BlockSpec(memory_space=pltpu.VMEM),
        ],
        out_specs=pl.BlockSpec(memory_space=pltpu.VMEM),
        scratch_shapes=[
            pltpu.VMEM((NSLOT, m_per, k), jnp.float32),
            pltpu.VMEM((NSLOT, m_per, k), jnp.float32),
            pltpu.SemaphoreType.DMA((NSLOT, 2)),
            pltpu.SemaphoreType.DMA((NSLOT, 2)),
            pltpu.SemaphoreType.DMA((NSLOT, 2)),
            pltpu.SemaphoreType.DMA((NSLOT, 2)),
            pltpu.SemaphoreType.REGULAR,
            pltpu.SemaphoreType.REGULAR,
        ],
        compiler_params=pltpu.CompilerParams(
            collective_id=0, vmem_limit_bytes=100 * 1024 * 1024
        ),
    )(x, w_mat)


# device time: 355348 ns/iter; 2.1439x vs baseline; 1.0060x over previous
import jax
import jax.numpy as jnp
from jax import lax
from jax.experimental import pallas as pl
from jax.experimental.pallas import tpu as pltpu

N_DEV = 16
HOPS = 8
HALF = 128
NSLOT = 4


def kernel(x, w_mat):
    m_per, k = x.shape
    _, n_per = w_mat.shape

    def body(x_ref, w_ref, out_ref,
             comm_r, comm_l,
             send_r, recv_r, send_l, recv_l,
             credit_r, credit_l):
        my = lax.axis_index("i")

        def m_of(u):
            u = lax.rem(u + N_DEV, N_DEV)
            zz = u // 4
            return 4 * zz + lax.rem(u % 4 - zz + 4, 4)

        z_pl = my // 4
        v_my = 4 * z_pl + lax.rem(my % 4 + z_pl, 4)
        left = m_of(v_my - 1)
        right = m_of(v_my + 1)

        def compute(origin, chunk, row0=0, rows=m_per):
            y = jnp.dot(chunk, w_ref[...], preferred_element_type=jnp.float32)
            out_ref[pl.ds(origin * m_per + row0, rows), :] = y * jax.nn.sigmoid(y)

        def pieces(j, dirn):
            if j < HOPS - 1:
                return (0, 1)
            return (0,) if dirn == 0 else (1,)

        def mk(j, p, dirn):
            comm = comm_r if dirn == 0 else comm_l
            rows = pl.ds(p * HALF, HALF)
            src = x_ref.at[rows] if j == 0 else comm.at[j % NSLOT, rows]
            return pltpu.make_async_remote_copy(
                src_ref=src,
                dst_ref=comm.at[(j + 1) % NSLOT, rows],
                send_sem=(send_r if dirn == 0 else send_l).at[j % NSLOT, p],
                recv_sem=(recv_r if dirn == 0 else recv_l).at[(j + 1) % NSLOT, p],
                device_id=(right,) if dirn == 0 else (left,),
                device_id_type=pl.DeviceIdType.MESH,
            )

        barrier_sem = pltpu.get_barrier_semaphore()
        for nbr in (left, right):
            pl.semaphore_signal(
                barrier_sem, inc=1,
                device_id=(nbr,), device_id_type=pl.DeviceIdType.MESH,
            )
        pl.semaphore_wait(barrier_sem, 2)

        descs = {}
        for j in range(HOPS):
            if j >= 1:
                for p in pieces(j - 1, 0):
                    descs[0, j - 1, p].wait_send()
                if 2 <= j <= 5:
                    pl.semaphore_signal(
                        credit_r, inc=1,
                        device_id=(left,), device_id_type=pl.DeviceIdType.MESH,
                    )
                for p in pieces(j - 1, 1):
                    descs[1, j - 1, p].wait_send()
                if 2 <= j <= 5:
                    pl.semaphore_signal(
                        credit_l, inc=1,
                        device_id=(right,), device_id_type=pl.DeviceIdType.MESH,
                    )
            if j >= 4:
                pl.semaphore_wait(credit_r, 1)
                pl.semaphore_wait(credit_l, 1)

            if 0 in pieces(j, 0):
                descs[0, j, 0] = mk(j, 0, 0)
                descs[0, j, 0].start()
            if 0 in pieces(j, 1):
                descs[1, j, 0] = mk(j, 0, 1)
                descs[1, j, 0].start()

            if j >= 1:
                descs[0, j - 1, 1].wait_recv()
            if 1 in pieces(j, 0):
                descs[0, j, 1] = mk(j, 1, 0)
                descs[0, j, 1].start()
            if j >= 1:
                descs[1, j - 1, 1].wait_recv()
            if 1 in pieces(j, 1):
                descs[1, j, 1] = mk(j, 1, 1)
                descs[1, j, 1].start()

            if j == 0:
                compute(my, x_ref[...])
            else:
                compute(m_of(v_my - j), comm_r[j % NSLOT])
                compute(m_of(v_my + j), comm_l[j % NSLOT])

            if 0 in pieces(j, 0):
                descs[0, j, 0].wait_recv()
                if j == HOPS - 1:
                    anti = m_of(v_my + N_DEV // 2)
                    compute(anti, comm_r[0, pl.ds(0, HALF)], row0=0, rows=HALF)
            if 0 in pieces(j, 1):
                descs[1, j, 0].wait_recv()

        descs[1, HOPS - 1, 1].wait_recv()
        compute(anti, comm_l[0, pl.ds(HALF, HALF)], row0=HALF, rows=HALF)
        descs[0, HOPS - 1, 0].wait_send()
        descs[1, HOPS - 1, 1].wait_send()

    return pl.pallas_call(
        body,
        out_shape=jax.ShapeDtypeStruct((N_DEV * m_per, n_per), jnp.float32),
        in_specs=[
            pl.BlockSpec(memory_space=pltpu.VMEM),
            pl.BlockSpec(memory_space=pltpu.VMEM),
        ],
        out_specs=pl.BlockSpec(memory_space=pltpu.VMEM),
        scratch_shapes=[
            pltpu.VMEM((NSLOT, m_per, k), jnp.float32),
            pltpu.VMEM((NSLOT, m_per, k), jnp.float32),
            pltpu.SemaphoreType.DMA((NSLOT, 2)),
            pltpu.SemaphoreType.DMA((NSLOT, 2)),
            pltpu.SemaphoreType.DMA((NSLOT, 2)),
            pltpu.SemaphoreType.DMA((NSLOT, 2)),
            pltpu.SemaphoreType.REGULAR,
            pltpu.SemaphoreType.REGULAR,
        ],
        compiler_params=pltpu.CompilerParams(
            collective_id=0, vmem_limit_bytes=100 * 1024 * 1024
        ),
    )(x, w_mat)
